# baseline (speedup 1.0000x reference)
import functools

import jax
import jax.numpy as jnp
from jax import lax
from jax.experimental import pallas as pl
from jax.experimental.pallas import tpu as pltpu
from jax.experimental.pallas import tpu_sc as plsc

N_NODES = 100000
D_FEAT = 128
NUM_TYPES = 64

CHUNK = 128
NBUF = 3
NMAIN = 15
SPAN = NMAIN * CHUNK

_INFO = plsc.get_sparse_core_info()
_NC = _INFO.num_cores
_NS = _INFO.num_subcores
_NW = _NC * _NS
N_SC = _NW * SPAN

N_TC = N_NODES - N_SC
TCB = 1928


def _sc_body(x_hbm, idx_hbm, tab_hbm, out_hbm, *scratch):
    tab_v = scratch[0]
    idx_all = scratch[1]
    x_v = scratch[2:2 + NBUF]
    sem_x = scratch[2 + NBUF:2 + 2 * NBUF]
    sem_o = scratch[2 + 2 * NBUF:2 + 3 * NBUF]

    wid = lax.axis_index("s") * _NC + lax.axis_index("c")
    span = wid * SPAN

    @pl.when(lax.axis_index("s") == 0)
    def _():
        pltpu.sync_copy(tab_hbm, tab_v)
    plsc.subcore_barrier()

    pltpu.sync_copy(idx_hbm.at[pl.ds(span, SPAN)], idx_all)

    def load(k, b):
        base = span + k * CHUNK
        pltpu.async_copy(x_hbm.at[pl.ds(base, CHUNK), :], x_v[b], sem_x[b])

    def wait_loads(b):
        pltpu.make_async_copy(x_hbm.at[pl.ds(0, CHUNK), :], x_v[b],
                              sem_x[b]).wait()

    def add_rows(b, k):
        pltpu.sync_copy(tab_v.at[idx_all.at[pl.ds(k * CHUNK, CHUNK)]],
                        x_v[b], add=True)

    def store(k, b):
        base = span + k * CHUNK
        pltpu.async_copy(x_v[b], out_hbm.at[pl.ds(base, CHUNK), :], sem_o[b])

    def wait_store(b):
        pltpu.make_async_copy(x_v[b], out_hbm.at[pl.ds(0, CHUNK), :],
                              sem_o[b]).wait()

    load(0, 0)

    def turn(j, carry):
        for b in range(NBUF):
            k = j * NBUF + b
            bn = (b + 1) % NBUF
            @pl.when(k + 1 < NMAIN)
            def _():
                @pl.when(k >= NBUF - 1)
                def _():
                    wait_store(bn)
                load(k + 1, bn)
            wait_loads(b)
            add_rows(b, k)
            store(k, b)
        return carry

    lax.fori_loop(0, NMAIN // NBUF, turn, 0, unroll=False)

    for b in range(NBUF):
        wait_store(b)


def _run_sc(x, idx, tab):
    mesh = plsc.VectorSubcoreMesh(core_axis_name="c", subcore_axis_name="s")
    f = pl.kernel(
        _sc_body,
        out_type=jax.ShapeDtypeStruct((N_SC, D_FEAT), jnp.float32),
        mesh=mesh,
        scratch_types=(
            [pltpu.VMEM_SHARED((NUM_TYPES, D_FEAT), jnp.float32)]
            + [pltpu.VMEM((SPAN,), jnp.int32)]
            + [pltpu.VMEM((CHUNK, D_FEAT), jnp.float32) for _ in range(NBUF)]
            + [pltpu.SemaphoreType.DMA for _ in range(2 * NBUF)]
        ),
    )
    return f(x, idx, tab)


def _tc_body(idx_ref, x_ref, tab_ref, out_ref):
    t = idx_ref[0, 0, :]
    onehot = (t[:, None] == lax.broadcasted_iota(jnp.int32, (1, NUM_TYPES), 1)
              ).astype(jnp.float32)
    out_ref[...] = x_ref[...] + jnp.dot(onehot, tab_ref[...],
                                        preferred_element_type=jnp.float32)


def _run_tc(x, idx, tab):
    nb = x.shape[0] // TCB
    idx3 = idx.reshape(nb, 1, TCB)
    return pl.pallas_call(
        _tc_body,
        grid=(nb,),
        in_specs=[
            pl.BlockSpec((1, 1, TCB), lambda i: (i, 0, 0)),
            pl.BlockSpec((TCB, D_FEAT), lambda i: (i, 0)),
            pl.BlockSpec((NUM_TYPES, D_FEAT), lambda i: (0, 0)),
        ],
        out_specs=pl.BlockSpec((TCB, D_FEAT), lambda i: (i, 0)),
        out_shape=jax.ShapeDtypeStruct((x.shape[0], D_FEAT), jnp.float32),
    )(idx3, x, tab)


@jax.jit
def _run(x, idx, tab):
    out_sc = _run_sc(x[:N_SC], idx[:N_SC], tab)
    out_tc = _run_tc(x[N_SC:], idx[N_SC:], tab)
    return jnp.concatenate([out_sc, out_tc], axis=0)


def kernel(x, node_types, type_table):
    idx = node_types.astype(jnp.int32)
    return _run(x, idx, type_table)


# speedup vs baseline: 2.1496x; 2.1496x over previous
import functools

import jax
import jax.numpy as jnp
from jax import lax
from jax.experimental import pallas as pl
from jax.experimental.pallas import tpu as pltpu
from jax.experimental.pallas import tpu_sc as plsc

N_NODES = 100000
D_FEAT = 128
NUM_TYPES = 64
CHUNK = 256
NBUF = 3
NMAIN = 12
BIG = NMAIN * CHUNK + 56
SMALL = NMAIN * CHUNK + 48
NBIG = 20
TAIL_B = 56
TAIL_S = 48

_INFO = plsc.get_sparse_core_info()
_NC = _INFO.num_cores
_NS = _INFO.num_subcores
_NW = _NC * _NS


def _sc_body(x_hbm, idx_hbm, tab_hbm, out_hbm, *scratch):
    tab_v = scratch[0]
    idx_all = scratch[1]
    x_v = scratch[2:2 + NBUF]
    sem_x = scratch[2 + NBUF:2 + 2 * NBUF]
    sem_o = scratch[2 + 2 * NBUF:2 + 3 * NBUF]

    wid = lax.axis_index("s") * _NC + lax.axis_index("c")
    span = wid * SMALL + jnp.minimum(wid, NBIG) * (TAIL_B - TAIL_S)

    @pl.when(lax.axis_index("s") == 0)
    def _():
        pltpu.sync_copy(tab_hbm, tab_v)
    plsc.subcore_barrier()

    @pl.when(wid < NBIG)
    def _():
        pltpu.sync_copy(idx_hbm.at[pl.ds(span, BIG)],
                        idx_all.at[pl.ds(0, BIG)])

    @pl.when(wid >= NBIG)
    def _():
        pltpu.sync_copy(idx_hbm.at[pl.ds(span, SMALL)],
                        idx_all.at[pl.ds(0, SMALL)])

    def load(k, b):
        base = span + k * CHUNK
        pltpu.async_copy(x_hbm.at[pl.ds(base, CHUNK), :], x_v[b], sem_x[b])

    def wait_loads(b):
        pltpu.make_async_copy(x_hbm.at[pl.ds(0, CHUNK), :], x_v[b],
                              sem_x[b]).wait()

    def add_rows(b, k):
        for h in range(CHUNK // 128):
            pltpu.sync_copy(
                tab_v.at[idx_all.at[pl.ds(k * CHUNK + h * 128, 128)]],
                x_v[b].at[pl.ds(h * 128, 128), :], add=True)

    def store(k, b):
        base = span + k * CHUNK
        pltpu.async_copy(x_v[b], out_hbm.at[pl.ds(base, CHUNK), :], sem_o[b])

    def wait_store(b):
        pltpu.make_async_copy(x_v[b], out_hbm.at[pl.ds(0, CHUNK), :],
                              sem_o[b]).wait()

    load(0, 0)

    def turn(j, carry):
        for b in range(NBUF):
            k = j * NBUF + b
            bn = (b + 1) % NBUF
            @pl.when(k + 1 < NMAIN)
            def _():
                @pl.when(k >= NBUF - 1)
                def _():
                    wait_store(bn)
                load(k + 1, bn)
            wait_loads(b)
            add_rows(b, k)
            store(k, b)
        return carry

    lax.fori_loop(0, NMAIN // NBUF, turn, 0, unroll=False)

    wait_store(0)
    tail_base = span + NMAIN * CHUNK

    @pl.when(wid < NBIG)
    def _():
        pltpu.async_copy(x_hbm.at[pl.ds(tail_base, TAIL_B), :],
                         x_v[0].at[pl.ds(0, TAIL_B), :], sem_x[0])

    @pl.when(wid >= NBIG)
    def _():
        pltpu.async_copy(x_hbm.at[pl.ds(tail_base, TAIL_S), :],
                         x_v[0].at[pl.ds(0, TAIL_S), :], sem_x[0])

    for b in range(1, NBUF):
        wait_store(b)

    def do_tail(nrows):
        base = span + NMAIN * CHUNK
        pltpu.make_async_copy(x_hbm.at[pl.ds(0, nrows), :],
                              x_v[0].at[pl.ds(0, nrows), :], sem_x[0]).wait()
        pltpu.sync_copy(tab_v.at[idx_all.at[pl.ds(NMAIN * CHUNK, nrows)]],
                        x_v[0].at[pl.ds(0, nrows), :], add=True)
        pltpu.sync_copy(x_v[0].at[pl.ds(0, nrows), :],
                        out_hbm.at[pl.ds(base, nrows), :])

    @pl.when(wid < NBIG)
    def _():
        do_tail(TAIL_B)

    @pl.when(wid >= NBIG)
    def _():
        do_tail(TAIL_S)


@jax.jit
def _run(x, idx, tab):
    mesh = plsc.VectorSubcoreMesh(core_axis_name="c", subcore_axis_name="s")
    f = pl.kernel(
        _sc_body,
        out_type=jax.ShapeDtypeStruct((N_NODES, D_FEAT), jnp.float32),
        mesh=mesh,
        scratch_types=(
            [pltpu.VMEM_SHARED((NUM_TYPES, D_FEAT), jnp.float32)]
            + [pltpu.VMEM((BIG,), jnp.int32)]
            + [pltpu.VMEM((CHUNK, D_FEAT), jnp.float32) for _ in range(NBUF)]
            + [pltpu.SemaphoreType.DMA for _ in range(2 * NBUF)]
        ),
    )
    return f(x, idx, tab)


def kernel(x, node_types, type_table):
    idx = node_types.astype(jnp.int32)
    return _run(x, idx, type_table)


# speedup vs baseline: 2.1576x; 1.0038x over previous
import jax
import jax.numpy as jnp
from jax import lax
from jax.experimental import pallas as pl
from jax.experimental.pallas import tpu as pltpu
from jax.experimental.pallas import tpu_sc as plsc

N_NODES = 100000
D_FEAT = 128
NUM_TYPES = 64
CHUNK = 256
NBUF = 3
NMAIN = 12
BIG = NMAIN * CHUNK + 56
SMALL = NMAIN * CHUNK + 48
NBIG = 20
TAIL_B = 56
TAIL_S = 48

_INFO = plsc.get_sparse_core_info()
_NC = _INFO.num_cores
_NS = _INFO.num_subcores
_NW = _NC * _NS


def _sc_body(x_hbm, idx_hbm, tab_hbm, out_hbm, *scratch):
    tab_v = scratch[0]
    idx_all = scratch[1]
    x_v = scratch[2:2 + NBUF]
    sem_x = scratch[2 + NBUF:2 + 2 * NBUF]
    sem_o = scratch[2 + 2 * NBUF:2 + 3 * NBUF]

    wid = lax.axis_index("s") * _NC + lax.axis_index("c")
    span = wid * SMALL + jnp.minimum(wid, NBIG) * (TAIL_B - TAIL_S)

    @pl.when(lax.axis_index("s") == 0)
    def _():
        pltpu.sync_copy(tab_hbm, tab_v)
    plsc.subcore_barrier()

    @pl.when(wid < NBIG)
    def _():
        pltpu.sync_copy(idx_hbm.at[pl.ds(span, BIG)],
                        idx_all.at[pl.ds(0, BIG)])

    @pl.when(wid >= NBIG)
    def _():
        pltpu.sync_copy(idx_hbm.at[pl.ds(span, SMALL)],
                        idx_all.at[pl.ds(0, SMALL)])

    def load(k, b):
        base = span + k * CHUNK
        pltpu.async_copy(x_hbm.at[pl.ds(base, CHUNK), :], x_v[b], sem_x[b])

    def wait_loads(b):
        pltpu.make_async_copy(x_hbm.at[pl.ds(0, CHUNK), :], x_v[b],
                              sem_x[b]).wait()

    def add_rows(b, k):
        for h in range(CHUNK // 128):
            pltpu.sync_copy(
                tab_v.at[idx_all.at[pl.ds(k * CHUNK + h * 128, 128)]],
                x_v[b].at[pl.ds(h * 128, 128), :], add=True)

    def store(k, b):
        base = span + k * CHUNK
        pltpu.async_copy(x_v[b], out_hbm.at[pl.ds(base, CHUNK), :], sem_o[b])

    def wait_store(b):
        pltpu.make_async_copy(x_v[b], out_hbm.at[pl.ds(0, CHUNK), :],
                              sem_o[b]).wait()

    load(0, 0)

    def turn(j, carry):
        for b in range(NBUF):
            k = j * NBUF + b
            bn = (b + 1) % NBUF
            @pl.when(k + 1 < NMAIN)
            def _():
                @pl.when(k >= NBUF - 1)
                def _():
                    wait_store(bn)
                load(k + 1, bn)
            wait_loads(b)
            add_rows(b, k)
            store(k, b)
        return carry

    lax.fori_loop(0, NMAIN // NBUF, turn, 0, unroll=False)

    wait_store(0)
    tail_base = span + NMAIN * CHUNK

    @pl.when(wid < NBIG)
    def _():
        pltpu.async_copy(x_hbm.at[pl.ds(tail_base, TAIL_B), :],
                         x_v[0].at[pl.ds(0, TAIL_B), :], sem_x[0])

    @pl.when(wid >= NBIG)
    def _():
        pltpu.async_copy(x_hbm.at[pl.ds(tail_base, TAIL_S), :],
                         x_v[0].at[pl.ds(0, TAIL_S), :], sem_x[0])

    for b in range(1, NBUF):
        wait_store(b)

    def do_tail(nrows):
        base = span + NMAIN * CHUNK
        pltpu.make_async_copy(x_hbm.at[pl.ds(0, nrows), :],
                              x_v[0].at[pl.ds(0, nrows), :], sem_x[0]).wait()
        pltpu.sync_copy(tab_v.at[idx_all.at[pl.ds(NMAIN * CHUNK, nrows)]],
                        x_v[0].at[pl.ds(0, nrows), :], add=True)
        pltpu.sync_copy(x_v[0].at[pl.ds(0, nrows), :],
                        out_hbm.at[pl.ds(base, nrows), :])

    @pl.when(wid < NBIG)
    def _():
        do_tail(TAIL_B)

    @pl.when(wid >= NBIG)
    def _():
        do_tail(TAIL_S)


@jax.jit
def _run(x, idx, tab):
    mesh = plsc.VectorSubcoreMesh(core_axis_name="c", subcore_axis_name="s")
    f = pl.kernel(
        _sc_body,
        out_type=jax.ShapeDtypeStruct((N_NODES, D_FEAT), jnp.float32),
        mesh=mesh,
        scratch_types=(
            [pltpu.VMEM_SHARED((NUM_TYPES, D_FEAT), jnp.float32)]
            + [pltpu.VMEM((BIG,), jnp.int32)]
            + [pltpu.VMEM((CHUNK, D_FEAT), jnp.float32) for _ in range(NBUF)]
            + [pltpu.SemaphoreType.DMA for _ in range(2 * NBUF)]
        ),
    )
    return f(x, idx, tab)


def kernel(x, node_types, type_table):
    idx = node_types.astype(jnp.int32)
    return _run(x, idx, type_table)
